# baseline (speedup 1.0000x reference)
import functools

import jax
import jax.numpy as jnp
from jax import lax
from jax.experimental import pallas as pl
from jax.experimental.pallas import tpu as pltpu
from jax.experimental.pallas import tpu_sc as plsc

N, E, D, O, B, NNEG = 10000, 320000, 128, 128, 8192, 5
NP = 10240
CH = 128
NCH = E // CH
RPT = NP // 16
SUBG = 128


def _tc_embed(xp, W, a_src2, a_dst2):
    nb = NP // 512

    def body(x_ref, w_ref, asr, adr, h_ref, ls_ref, ld_ref):
        h = jnp.dot(x_ref[...], w_ref[...], preferred_element_type=jnp.float32)
        h_ref[...] = h
        ls_ref[0, 0, :] = jnp.sum(h * asr[...], axis=1)
        ld_ref[0, 0, :] = jnp.sum(h * adr[...], axis=1)

    return pl.pallas_call(
        body,
        grid=(nb,),
        in_specs=[
            pl.BlockSpec((512, D), lambda i: (i, 0)),
            pl.BlockSpec((D, O), lambda i: (0, 0)),
            pl.BlockSpec((1, O), lambda i: (0, 0)),
            pl.BlockSpec((1, O), lambda i: (0, 0)),
        ],
        out_specs=[
            pl.BlockSpec((512, O), lambda i: (i, 0)),
            pl.BlockSpec((1, 1, 512), lambda i: (i, 0, 0)),
            pl.BlockSpec((1, 1, 512), lambda i: (i, 0, 0)),
        ],
        out_shape=[
            jax.ShapeDtypeStruct((NP, O), jnp.float32),
            jax.ShapeDtypeStruct((nb, 1, 512), jnp.float32),
            jax.ShapeDtypeStruct((nb, 1, 512), jnp.float32),
        ],
    )(xp, W, a_src2, a_dst2)


def _lane(v, t):
    return lax.squeeze(lax.slice_in_dim(v, t, t + 1), (0,))


def _sc_gat(h, ls, ld, es, ed):
    mesh = plsc.VectorSubcoreMesh(core_axis_name="c", subcore_axis_name="s")
    cpt = NCH // 32
    nleft = NCH - 32 * cpt
    npair = cpt // 2

    ph0 = 40
    ph1 = cpt - ph0

    @functools.partial(
        pl.kernel,
        out_type=[
            jax.ShapeDtypeStruct((2, NP, O), jnp.float32),
            jax.ShapeDtypeStruct((2, NP), jnp.float32),
        ],
        mesh=mesh,
        scratch_types=[
            pltpu.VMEM_SHARED((NP, O), jnp.float32),
            pltpu.VMEM_SHARED((NP,), jnp.float32),
            pltpu.VMEM((ph0, 1, CH), jnp.int32),
            pltpu.VMEM((ph0, 1, CH), jnp.int32),
            [pltpu.VMEM((CH, O), jnp.float32) for _ in range(2)],
            [pltpu.VMEM((CH,), jnp.float32) for _ in range(2)],
            [pltpu.VMEM((CH,), jnp.float32) for _ in range(2)],
            [pltpu.VMEM((CH,), jnp.float32) for _ in range(2)],
            [pltpu.SemaphoreType.DMA for _ in range(2)],
            [pltpu.SemaphoreType.DMA for _ in range(2)],
        ],
    )
    def k(h_hbm, ls_hbm, ld_hbm, es_hbm, ed_hbm, acc_hbm, den_hbm,
          out_sh, den_sh, sixb, dixb, rows, exv, lsg, ldg, sem_b, sem_d):
        c = lax.axis_index("c")
        s = lax.axis_index("s")
        wid = c * 16 + s
        zero16 = jnp.zeros((16,), jnp.float32)

        def zrow(r, _):
            for kk in range(O // 16):
                rows[0][r, pl.ds(kk * 16, 16)] = zero16
            return 0

        lax.fori_loop(0, CH, zrow, 0)
        for j in range(CH // 16):
            exv[0][pl.ds(j * 16, 16)] = zero16
        r0 = pl.multiple_of(s * RPT, 128)
        for m in range(RPT // CH):
            pltpu.sync_copy(rows[0], out_sh.at[pl.ds(r0 + m * CH, CH)])
            pltpu.sync_copy(exv[0], den_sh.at[pl.ds(r0 + m * CH, CH)])
        plsc.subcore_barrier()

        cb = wid * cpt

        def fire_b(p, jj):
            pltpu.async_copy(h_hbm.at[sixb.at[jj, 0]], rows[p], sem_b[p])
            pltpu.async_copy(ls_hbm.at[sixb.at[jj, 0]], lsg[p], sem_b[p])
            pltpu.async_copy(ld_hbm.at[dixb.at[jj, 0]], ldg[p], sem_b[p])

        def drain_b(p, jj):
            pltpu.make_async_copy(h_hbm.at[sixb.at[jj, 0]], rows[p],
                                  sem_b[p]).wait()
            pltpu.make_async_copy(ls_hbm.at[sixb.at[jj, 0]], lsg[p],
                                  sem_b[p]).wait()
            pltpu.make_async_copy(ld_hbm.at[dixb.at[jj, 0]], ldg[p],
                                  sem_b[p]).wait()

        def compute(p, j0, j1):
            def jbody(j, _):
                j16 = j * 16
                lg = lsg[p][pl.ds(j16, 16)] + ldg[p][pl.ds(j16, 16)]
                e = jnp.maximum(lg, jnp.float32(0.2) * lg)
                ex = jnp.exp(e)
                exv[p][pl.ds(j16, 16)] = ex
                for t in range(16):
                    r = j16 + t
                    w = jnp.full((16,), _lane(ex, t))
                    for kk in range(O // 16):
                        rows[p][r, pl.ds(kk * 16, 16)] = (
                            rows[p][r, pl.ds(kk * 16, 16)] * w)
                return 0

            lax.fori_loop(j0, j1, jbody, 0)

        def fire_d(p, jj):
            pltpu.async_copy(rows[p], out_sh.at[dixb.at[jj, 0]], sem_d[p],
                             add=True)
            pltpu.async_copy(exv[p], den_sh.at[dixb.at[jj, 0]], sem_d[p],
                             add=True)

        def drain_d(p, jj):
            pltpu.make_async_copy(rows[p], out_sh.at[dixb.at[jj, 0]],
                                  sem_d[p]).wait()
            pltpu.make_async_copy(exv[p], den_sh.at[dixb.at[jj, 0]],
                                  sem_d[p]).wait()

        def run_phase(c0, nrows):
            pltpu.sync_copy(es_hbm.at[pl.ds(c0, nrows)],
                            sixb.at[pl.ds(0, nrows)])
            pltpu.sync_copy(ed_hbm.at[pl.ds(c0, nrows)],
                            dixb.at[pl.ds(0, nrows)])
            nv = nrows - 1

            hf = CH // 32

            def pair(i2, first):
                a = 2 * i2
                b = a + 1
                nxt = jnp.minimum(2 * i2 + 2, nv)
                drain_b(0, a)
                compute(0, 0, hf)
                if not first:
                    drain_d(1, b - 2)
                fire_b(1, b)
                compute(0, hf, 2 * hf)
                fire_d(0, a)
                drain_b(1, b)
                compute(1, 0, hf)
                drain_d(0, a)
                fire_b(0, nxt)
                compute(1, hf, 2 * hf)
                fire_d(1, b)

            fire_b(0, 0)
            pair(0, True)

            def pbody(i2, _):
                pair(i2, False)
                return 0

            lax.fori_loop(1, nrows // 2, pbody, 0)
            drain_b(0, nv)
            drain_d(1, nv)

        run_phase(cb, ph0)
        run_phase(cb + ph0, ph1)

        @pl.when(wid < nleft)
        def _():
            ci = NCH - nleft + wid
            pltpu.sync_copy(es_hbm.at[pl.ds(ci, 1)], sixb.at[pl.ds(0, 1)])
            pltpu.sync_copy(ed_hbm.at[pl.ds(ci, 1)], dixb.at[pl.ds(0, 1)])
            pltpu.sync_copy(h_hbm.at[sixb.at[0, 0]], rows[0])
            pltpu.sync_copy(ls_hbm.at[sixb.at[0, 0]], lsg[0])
            pltpu.sync_copy(ld_hbm.at[dixb.at[0, 0]], ldg[0])
            compute(0, 0, CH // 16)
            pltpu.sync_copy(rows[0], out_sh.at[dixb.at[0, 0]], add=True)
            pltpu.sync_copy(exv[0], den_sh.at[dixb.at[0, 0]], add=True)

        plsc.subcore_barrier()

        pltpu.sync_copy(out_sh.at[pl.ds(r0, RPT)],
                        acc_hbm.at[c].at[pl.ds(r0, RPT)])
        pltpu.sync_copy(den_sh.at[pl.ds(r0, RPT)],
                        den_hbm.at[c].at[pl.ds(r0, RPT)])

    return k(h, ls, ld, es, ed)


def _tc_combine(acc, den):
    nb = NP // 512

    def body(a_ref, d_ref, e_ref):
        dn = d_ref[0, 0, 0] + d_ref[1, 0, 0] + jnp.float32(1e-16)
        v = (a_ref[0] + a_ref[1]) / dn.reshape(512, 1)
        e_ref[...] = jnp.where(v > 0, v, jnp.exp(v) - jnp.float32(1.0))

    return pl.pallas_call(
        body,
        grid=(nb,),
        in_specs=[
            pl.BlockSpec((2, 512, O), lambda i: (0, i, 0)),
            pl.BlockSpec((2, 1, 1, 512), lambda i: (0, i, 0, 0)),
        ],
        out_specs=pl.BlockSpec((512, O), lambda i: (i, 0)),
        out_shape=jax.ShapeDtypeStruct((NP, O), jnp.float32),
    )(acc, den.reshape(2, NP // 512, 1, 512))


def _sc_gather(emb, si, pi, nit):
    mesh = plsc.VectorSubcoreMesh(core_axis_name="c", subcore_axis_name="s")
    gpt = B // 32

    @functools.partial(
        pl.kernel,
        out_type=[
            jax.ShapeDtypeStruct((B, O), jnp.float32),
            jax.ShapeDtypeStruct((B, O), jnp.float32),
            jax.ShapeDtypeStruct((NNEG * B, O), jnp.float32),
        ],
        mesh=mesh,
        scratch_types=[
            pltpu.VMEM((2 * SUBG,), jnp.int32),
            pltpu.VMEM((2 * SUBG,), jnp.int32),
            pltpu.VMEM((NNEG, 1, 2 * SUBG), jnp.int32),
            [pltpu.VMEM((SUBG, O), jnp.float32) for _ in range(7)],
            [pltpu.SemaphoreType.DMA for _ in range(7)],
        ],
    )
    def k(emb_hbm, si_hbm, pi_hbm, ni_hbm, srcE, posE, negE,
          sidx, pidx, nidx, bufs, sem):
        c = lax.axis_index("c")
        s = lax.axis_index("s")
        wid = c * 16 + s
        gb = pl.multiple_of(wid * gpt, 128)
        pltpu.sync_copy(si_hbm.at[pl.ds(gb, 2 * SUBG)], sidx)
        pltpu.sync_copy(pi_hbm.at[pl.ds(gb, 2 * SUBG)], pidx)
        for kk in range(NNEG):
            pltpu.sync_copy(ni_hbm.at[kk, 0, pl.ds(gb, 2 * SUBG)],
                            nidx.at[kk, 0])

        def sd(u):
            o = u * SUBG
            g0 = pl.multiple_of(wid * gpt + o, 128)
            srcs = ([emb_hbm.at[sidx.at[pl.ds(o, SUBG)]],
                     emb_hbm.at[pidx.at[pl.ds(o, SUBG)]]]
                    + [emb_hbm.at[nidx.at[kk, 0, pl.ds(o, SUBG)]]
                       for kk in range(NNEG)])
            dsts = ([srcE.at[pl.ds(g0, SUBG)], posE.at[pl.ds(g0, SUBG)]]
                    + [negE.at[pl.ds(kk * B + g0, SUBG)]
                       for kk in range(NNEG)])
            return srcs, dsts

        for u in range(gpt // SUBG):
            srcs, dsts = sd(u)
            if u > 0:
                psrcs, pdsts = sd(u - 1)
            for q in range(7):
                if u > 0:
                    pltpu.make_async_copy(bufs[q], pdsts[q], sem[q]).wait()
                pltpu.async_copy(srcs[q], bufs[q], sem[q])
            for q in range(7):
                pltpu.make_async_copy(srcs[q], bufs[q], sem[q]).wait()
                pltpu.async_copy(bufs[q], dsts[q], sem[q])
        srcs, dsts = sd(gpt // SUBG - 1)
        for q in range(7):
            pltpu.make_async_copy(bufs[q], dsts[q], sem[q]).wait()

    return k(emb, si, pi, nit)


def _tc_loss(srcE, posE, negE):
    bs = 512
    nb = B // bs

    def body(s_ref, p_ref, n_ref, o_ref):
        eps = jnp.float32(1e-16)
        i = pl.program_id(0)
        sv = s_ref[...]
        pv = p_ref[...]
        ns = jnp.sum(sv * sv, axis=1)
        npn = jnp.sum(pv * pv, axis=1)
        dp = jnp.sum(sv * pv, axis=1)
        cp = dp * lax.rsqrt(jnp.maximum(ns * npn, eps))
        hin = jnp.zeros((bs,), jnp.float32)
        for kk in range(NNEG):
            nk = n_ref[kk]
            nn = jnp.sum(nk * nk, axis=1)
            dn = jnp.sum(sv * nk, axis=1)
            cn = dn * lax.rsqrt(jnp.maximum(ns * nn, eps))
            hin = hin + jnp.maximum(cn - cp + jnp.float32(1.0),
                                    jnp.float32(0.0))
        ps = jnp.sum(hin)
        mask = ((lax.broadcasted_iota(jnp.int32, (8, 128), 0) == 0)
                & (lax.broadcasted_iota(jnp.int32, (8, 128), 1) == 0))

        @pl.when(i == 0)
        def _():
            o_ref[...] = jnp.zeros((8, 128), jnp.float32)

        o_ref[...] = o_ref[...] + jnp.where(mask, ps, jnp.float32(0.0))

    return pl.pallas_call(
        body,
        grid=(nb,),
        in_specs=[
            pl.BlockSpec((bs, O), lambda i: (i, 0)),
            pl.BlockSpec((bs, O), lambda i: (i, 0)),
            pl.BlockSpec((NNEG, bs, O), lambda i: (0, i, 0)),
        ],
        out_specs=pl.BlockSpec((8, 128), lambda i: (0, 0)),
        out_shape=jax.ShapeDtypeStruct((8, 128), jnp.float32),
    )(srcE, posE, negE)


def kernel(x, adj, src_index, dst_pos_index, dst_neg_index, W, a_src, a_dst):
    xp = jnp.pad(x, ((0, NP - N), (0, 0)))
    h, ls3, ld3 = _tc_embed(xp, W, a_src.reshape(1, O), a_dst.reshape(1, O))
    ls = ls3.reshape(NP)
    ld = ld3.reshape(NP)
    acc, den = _sc_gat(h, ls, ld, adj[0].reshape(NCH, 1, CH),
                       adj[1].reshape(NCH, 1, CH))
    emb = _tc_combine(acc, den)
    nit = dst_neg_index.reshape(B, NNEG).T.reshape(NNEG, 1, B)
    srcE, posE, negE = _sc_gather(emb, src_index, dst_pos_index, nit)
    out = _tc_loss(srcE, posE, negE.reshape(NNEG, B, O))
    return out[0, 0] / jnp.float32(B * NNEG)


# speedup vs baseline: 1.0972x; 1.0972x over previous
import functools

import jax
import jax.numpy as jnp
from jax import lax
from jax.experimental import pallas as pl
from jax.experimental.pallas import tpu as pltpu
from jax.experimental.pallas import tpu_sc as plsc

N, E, D, O, B, NNEG = 10000, 320000, 128, 128, 8192, 5
NP = 10240
CH = 128
NCH = E // CH
RPT = NP // 16
SUBG = 128


def _tc_embed(xp, W, a_src2, a_dst2):
    nb = NP // 512

    def body(x_ref, w_ref, asr, adr, h_ref, ls_ref, ld_ref):
        h = jnp.dot(x_ref[...], w_ref[...], preferred_element_type=jnp.float32)
        h_ref[...] = h
        ls_ref[0, 0, :] = jnp.sum(h * asr[...], axis=1)
        ld_ref[0, 0, :] = jnp.sum(h * adr[...], axis=1)

    return pl.pallas_call(
        body,
        grid=(nb,),
        in_specs=[
            pl.BlockSpec((512, D), lambda i: (i, 0)),
            pl.BlockSpec((D, O), lambda i: (0, 0)),
            pl.BlockSpec((1, O), lambda i: (0, 0)),
            pl.BlockSpec((1, O), lambda i: (0, 0)),
        ],
        out_specs=[
            pl.BlockSpec((512, O), lambda i: (i, 0)),
            pl.BlockSpec((1, 1, 512), lambda i: (i, 0, 0)),
            pl.BlockSpec((1, 1, 512), lambda i: (i, 0, 0)),
        ],
        out_shape=[
            jax.ShapeDtypeStruct((NP, O), jnp.float32),
            jax.ShapeDtypeStruct((nb, 1, 512), jnp.float32),
            jax.ShapeDtypeStruct((nb, 1, 512), jnp.float32),
        ],
    )(xp, W, a_src2, a_dst2)


def _lane(v, t):
    return lax.squeeze(lax.slice_in_dim(v, t, t + 1), (0,))


def _sc_gat(h, ls, ld, es, ed):
    mesh = plsc.VectorSubcoreMesh(core_axis_name="c", subcore_axis_name="s")
    cpt = NCH // 32
    nleft = NCH - 32 * cpt
    npair = cpt // 2

    ph0 = 40
    ph1 = cpt - ph0

    @functools.partial(
        pl.kernel,
        out_type=[
            jax.ShapeDtypeStruct((2, NP, O), jnp.float32),
            jax.ShapeDtypeStruct((2, NP), jnp.float32),
        ],
        mesh=mesh,
        scratch_types=[
            pltpu.VMEM_SHARED((NP, O), jnp.float32),
            pltpu.VMEM_SHARED((NP,), jnp.float32),
            pltpu.VMEM((ph0, 1, CH), jnp.int32),
            pltpu.VMEM((ph0, 1, CH), jnp.int32),
            [pltpu.VMEM((CH, O), jnp.float32) for _ in range(2)],
            [pltpu.VMEM((CH,), jnp.float32) for _ in range(2)],
            [pltpu.VMEM((CH,), jnp.float32) for _ in range(2)],
            [pltpu.VMEM((CH,), jnp.float32) for _ in range(2)],
            [pltpu.SemaphoreType.DMA for _ in range(2)],
            [pltpu.SemaphoreType.DMA for _ in range(2)],
        ],
    )
    def k(h_hbm, ls_hbm, ld_hbm, es_hbm, ed_hbm, acc_hbm, den_hbm,
          out_sh, den_sh, sixb, dixb, rows, exv, lsg, ldg, sem_b, sem_d):
        c = lax.axis_index("c")
        s = lax.axis_index("s")
        wid = c * 16 + s
        zero16 = jnp.zeros((16,), jnp.float32)

        def zrow(r, _):
            for kk in range(O // 16):
                rows[0][r, pl.ds(kk * 16, 16)] = zero16
            return 0

        lax.fori_loop(0, CH, zrow, 0)
        for j in range(CH // 16):
            exv[0][pl.ds(j * 16, 16)] = zero16
        r0 = pl.multiple_of(s * RPT, 128)
        for m in range(RPT // CH):
            pltpu.sync_copy(rows[0], out_sh.at[pl.ds(r0 + m * CH, CH)])
            pltpu.sync_copy(exv[0], den_sh.at[pl.ds(r0 + m * CH, CH)])
        plsc.subcore_barrier()

        cb = wid * cpt

        def fire_b(p, jj):
            pltpu.async_copy(h_hbm.at[sixb.at[jj, 0]], rows[p], sem_b[p])
            pltpu.async_copy(ls_hbm.at[sixb.at[jj, 0]], lsg[p], sem_b[p])
            pltpu.async_copy(ld_hbm.at[dixb.at[jj, 0]], ldg[p], sem_b[p])

        def drain_b(p, jj):
            pltpu.make_async_copy(h_hbm.at[sixb.at[jj, 0]], rows[p],
                                  sem_b[p]).wait()
            pltpu.make_async_copy(ls_hbm.at[sixb.at[jj, 0]], lsg[p],
                                  sem_b[p]).wait()
            pltpu.make_async_copy(ld_hbm.at[dixb.at[jj, 0]], ldg[p],
                                  sem_b[p]).wait()

        def compute(p, j0, j1):
            def jbody(j, _):
                j16 = j * 16
                lg = lsg[p][pl.ds(j16, 16)] + ldg[p][pl.ds(j16, 16)]
                e = jnp.maximum(lg, jnp.float32(0.2) * lg)
                ex = jnp.exp(e)
                exv[p][pl.ds(j16, 16)] = ex
                for t in range(16):
                    r = j16 + t
                    w = jnp.full((16,), _lane(ex, t))
                    for kk in range(O // 16):
                        rows[p][r, pl.ds(kk * 16, 16)] = (
                            rows[p][r, pl.ds(kk * 16, 16)] * w)
                return 0

            lax.fori_loop(j0, j1, jbody, 0)

        def fire_d(p, jj):
            pltpu.async_copy(rows[p], out_sh.at[dixb.at[jj, 0]], sem_d[p],
                             add=True)
            pltpu.async_copy(exv[p], den_sh.at[dixb.at[jj, 0]], sem_d[p],
                             add=True)

        def drain_d(p, jj):
            pltpu.make_async_copy(rows[p], out_sh.at[dixb.at[jj, 0]],
                                  sem_d[p]).wait()
            pltpu.make_async_copy(exv[p], den_sh.at[dixb.at[jj, 0]],
                                  sem_d[p]).wait()

        def run_phase(c0, nrows):
            pltpu.sync_copy(es_hbm.at[pl.ds(c0, nrows)],
                            sixb.at[pl.ds(0, nrows)])
            pltpu.sync_copy(ed_hbm.at[pl.ds(c0, nrows)],
                            dixb.at[pl.ds(0, nrows)])
            nv = nrows - 1

            nj = CH // 16

            def pair(i2, first):
                a = 2 * i2
                b = a + 1
                nxt = jnp.minimum(2 * i2 + 2, nv)
                drain_b(0, a)
                if not first:
                    drain_d(1, b - 2)
                fire_b(1, b)
                compute(0, 0, nj)
                fire_d(0, a)
                drain_b(1, b)
                drain_d(0, a)
                fire_b(0, nxt)
                compute(1, 0, nj)
                fire_d(1, b)

            fire_b(0, 0)
            pair(0, True)

            def pbody(i2, _):
                pair(i2, False)
                return 0

            lax.fori_loop(1, nrows // 2, pbody, 0)
            drain_b(0, nv)
            drain_d(1, nv)

        run_phase(cb, ph0)
        run_phase(cb + ph0, ph1)

        @pl.when(wid < nleft)
        def _():
            ci = NCH - nleft + wid
            pltpu.sync_copy(es_hbm.at[pl.ds(ci, 1)], sixb.at[pl.ds(0, 1)])
            pltpu.sync_copy(ed_hbm.at[pl.ds(ci, 1)], dixb.at[pl.ds(0, 1)])
            pltpu.sync_copy(h_hbm.at[sixb.at[0, 0]], rows[0])
            pltpu.sync_copy(ls_hbm.at[sixb.at[0, 0]], lsg[0])
            pltpu.sync_copy(ld_hbm.at[dixb.at[0, 0]], ldg[0])
            compute(0, 0, CH // 16)
            pltpu.sync_copy(rows[0], out_sh.at[dixb.at[0, 0]], add=True)
            pltpu.sync_copy(exv[0], den_sh.at[dixb.at[0, 0]], add=True)

        plsc.subcore_barrier()

        pltpu.sync_copy(out_sh.at[pl.ds(r0, RPT)],
                        acc_hbm.at[c].at[pl.ds(r0, RPT)])
        pltpu.sync_copy(den_sh.at[pl.ds(r0, RPT)],
                        den_hbm.at[c].at[pl.ds(r0, RPT)])

    return k(h, ls, ld, es, ed)


def _tc_combine(acc, den):
    nb = NP // 512

    def body(a_ref, d_ref, e_ref):
        dn = d_ref[0, 0, 0] + d_ref[1, 0, 0] + jnp.float32(1e-16)
        v = (a_ref[0] + a_ref[1]) / dn.reshape(512, 1)
        e_ref[...] = jnp.where(v > 0, v, jnp.exp(v) - jnp.float32(1.0))

    return pl.pallas_call(
        body,
        grid=(nb,),
        in_specs=[
            pl.BlockSpec((2, 512, O), lambda i: (0, i, 0)),
            pl.BlockSpec((2, 1, 1, 512), lambda i: (0, i, 0, 0)),
        ],
        out_specs=pl.BlockSpec((512, O), lambda i: (i, 0)),
        out_shape=jax.ShapeDtypeStruct((NP, O), jnp.float32),
    )(acc, den.reshape(2, NP // 512, 1, 512))


def _sc_gather(emb, si, pi, nit):
    mesh = plsc.VectorSubcoreMesh(core_axis_name="c", subcore_axis_name="s")
    gpt = B // 32

    @functools.partial(
        pl.kernel,
        out_type=[
            jax.ShapeDtypeStruct((B, O), jnp.float32),
            jax.ShapeDtypeStruct((B, O), jnp.float32),
            jax.ShapeDtypeStruct((NNEG * B, O), jnp.float32),
        ],
        mesh=mesh,
        scratch_types=[
            pltpu.VMEM((2 * SUBG,), jnp.int32),
            pltpu.VMEM((2 * SUBG,), jnp.int32),
            pltpu.VMEM((NNEG, 1, 2 * SUBG), jnp.int32),
            [pltpu.VMEM((SUBG, O), jnp.float32) for _ in range(7)],
            [pltpu.SemaphoreType.DMA for _ in range(7)],
        ],
    )
    def k(emb_hbm, si_hbm, pi_hbm, ni_hbm, srcE, posE, negE,
          sidx, pidx, nidx, bufs, sem):
        c = lax.axis_index("c")
        s = lax.axis_index("s")
        wid = c * 16 + s
        gb = pl.multiple_of(wid * gpt, 128)
        pltpu.sync_copy(si_hbm.at[pl.ds(gb, 2 * SUBG)], sidx)
        pltpu.sync_copy(pi_hbm.at[pl.ds(gb, 2 * SUBG)], pidx)
        for kk in range(NNEG):
            pltpu.sync_copy(ni_hbm.at[kk, 0, pl.ds(gb, 2 * SUBG)],
                            nidx.at[kk, 0])

        def sd(u):
            o = u * SUBG
            g0 = pl.multiple_of(wid * gpt + o, 128)
            srcs = ([emb_hbm.at[sidx.at[pl.ds(o, SUBG)]],
                     emb_hbm.at[pidx.at[pl.ds(o, SUBG)]]]
                    + [emb_hbm.at[nidx.at[kk, 0, pl.ds(o, SUBG)]]
                       for kk in range(NNEG)])
            dsts = ([srcE.at[pl.ds(g0, SUBG)], posE.at[pl.ds(g0, SUBG)]]
                    + [negE.at[pl.ds(kk * B + g0, SUBG)]
                       for kk in range(NNEG)])
            return srcs, dsts

        for u in range(gpt // SUBG):
            srcs, dsts = sd(u)
            if u > 0:
                psrcs, pdsts = sd(u - 1)
            for q in range(7):
                if u > 0:
                    pltpu.make_async_copy(bufs[q], pdsts[q], sem[q]).wait()
                pltpu.async_copy(srcs[q], bufs[q], sem[q])
            for q in range(7):
                pltpu.make_async_copy(srcs[q], bufs[q], sem[q]).wait()
                pltpu.async_copy(bufs[q], dsts[q], sem[q])
        srcs, dsts = sd(gpt // SUBG - 1)
        for q in range(7):
            pltpu.make_async_copy(bufs[q], dsts[q], sem[q]).wait()

    return k(emb, si, pi, nit)


def _tc_loss(srcE, posE, negE):
    bs = 512
    nb = B // bs

    def body(s_ref, p_ref, n_ref, o_ref):
        eps = jnp.float32(1e-16)
        i = pl.program_id(0)
        sv = s_ref[...]
        pv = p_ref[...]
        ns = jnp.sum(sv * sv, axis=1)
        npn = jnp.sum(pv * pv, axis=1)
        dp = jnp.sum(sv * pv, axis=1)
        cp = dp * lax.rsqrt(jnp.maximum(ns * npn, eps))
        hin = jnp.zeros((bs,), jnp.float32)
        for kk in range(NNEG):
            nk = n_ref[kk]
            nn = jnp.sum(nk * nk, axis=1)
            dn = jnp.sum(sv * nk, axis=1)
            cn = dn * lax.rsqrt(jnp.maximum(ns * nn, eps))
            hin = hin + jnp.maximum(cn - cp + jnp.float32(1.0),
                                    jnp.float32(0.0))
        ps = jnp.sum(hin)
        mask = ((lax.broadcasted_iota(jnp.int32, (8, 128), 0) == 0)
                & (lax.broadcasted_iota(jnp.int32, (8, 128), 1) == 0))

        @pl.when(i == 0)
        def _():
            o_ref[...] = jnp.zeros((8, 128), jnp.float32)

        o_ref[...] = o_ref[...] + jnp.where(mask, ps, jnp.float32(0.0))

    return pl.pallas_call(
        body,
        grid=(nb,),
        in_specs=[
            pl.BlockSpec((bs, O), lambda i: (i, 0)),
            pl.BlockSpec((bs, O), lambda i: (i, 0)),
            pl.BlockSpec((NNEG, bs, O), lambda i: (0, i, 0)),
        ],
        out_specs=pl.BlockSpec((8, 128), lambda i: (0, 0)),
        out_shape=jax.ShapeDtypeStruct((8, 128), jnp.float32),
    )(srcE, posE, negE)


def kernel(x, adj, src_index, dst_pos_index, dst_neg_index, W, a_src, a_dst):
    xp = jnp.pad(x, ((0, NP - N), (0, 0)))
    h, ls3, ld3 = _tc_embed(xp, W, a_src.reshape(1, O), a_dst.reshape(1, O))
    ls = ls3.reshape(NP)
    ld = ld3.reshape(NP)
    acc, den = _sc_gat(h, ls, ld, adj[0].reshape(NCH, 1, CH),
                       adj[1].reshape(NCH, 1, CH))
    emb = _tc_combine(acc, den)
    nit = dst_neg_index.reshape(B, NNEG).T.reshape(NNEG, 1, B)
    srcE, posE, negE = _sc_gather(emb, src_index, dst_pos_index, nit)
    out = _tc_loss(srcE, posE, negE.reshape(NNEG, B, O))
    return out[0, 0] / jnp.float32(B * NNEG)
